# baseline (speedup 1.0000x reference)
import functools

import jax
import jax.numpy as jnp
from jax.experimental import pallas as pl
from jax.experimental.pallas import tpu as pltpu

N = 50000
D = 256
NZ_PAD = 128
S = 1024
BN = 2000
G = N // BN


def _prologue_body(emb_ref, w1a_ref, b1_ref, m_ref):
    m_ref[...] = (
        jnp.dot(emb_ref[...], w1a_ref[...], preferred_element_type=jnp.float32)
        + b1_ref[...]
    ).astype(jnp.bfloat16)


def _main_body(pos_ref, an_ref, batch_ref, m_ref, w1b_ref, w2_ref, b2_ref,
               w3_ref, b3_ref, out_ref):
    an = an_ref[...]
    onehot_an = (an == jax.lax.broadcasted_iota(jnp.int32, (1, NZ_PAD), 1)
                 ).astype(jnp.bfloat16)
    pre1 = (
        jnp.dot(onehot_an, m_ref[...], preferred_element_type=jnp.float32)
        + jnp.dot(pos_ref[...], w1b_ref[...], preferred_element_type=jnp.float32)
    )
    x1 = (0.5 * pre1) * (1.0 + jnp.tanh(0.5 * pre1))
    pre2 = jnp.dot(x1.astype(jnp.bfloat16), w2_ref[...],
                   preferred_element_type=jnp.float32) + b2_ref[...]
    x2 = (0.5 * pre2) * (1.0 + jnp.tanh(0.5 * pre2))
    e = jnp.dot(x2, w3_ref[...], preferred_element_type=jnp.float32) + b3_ref[...]

    seg = batch_ref[...]
    onehot_seg = (seg == jax.lax.broadcasted_iota(jnp.int32, (1, S), 1)
                  ).astype(jnp.float32)
    out_ref[0] = jax.lax.dot_general(
        e, onehot_seg, dimension_numbers=(((0,), (0,)), ((), ())),
        preferred_element_type=jnp.float32)


def _epilogue_body(partials_ref, out_ref):
    out_ref[...] = jnp.sum(partials_ref[...], axis=0)


@functools.partial(jax.jit, static_argnames=())
def kernel(pos, emb, W1, b1, W2, b2, W3, b3, atomic_numbers, batch):
    pos_pad = jnp.pad(pos.astype(jnp.float32), ((0, 0), (0, 5)))
    emb_pad = jnp.pad(emb, ((0, NZ_PAD - emb.shape[0]), (0, 0)))
    W1a = W1[:D, :]
    W1b = jnp.pad(W1[D:, :], ((0, 5), (0, 0)))
    an2d = atomic_numbers.astype(jnp.int32).reshape(N, 1)
    batch2d = batch.astype(jnp.int32).reshape(N, 1)

    M = pl.pallas_call(
        _prologue_body,
        out_shape=jax.ShapeDtypeStruct((NZ_PAD, D), jnp.bfloat16),
    )(emb_pad, W1a, b1.reshape(1, D))

    partials = pl.pallas_call(
        _main_body,
        grid=(G,),
        in_specs=[
            pl.BlockSpec((BN, 8), lambda i: (i, 0)),
            pl.BlockSpec((BN, 1), lambda i: (i, 0)),
            pl.BlockSpec((BN, 1), lambda i: (i, 0)),
            pl.BlockSpec((NZ_PAD, D), lambda i: (0, 0)),
            pl.BlockSpec((8, D), lambda i: (0, 0)),
            pl.BlockSpec((D, D), lambda i: (0, 0)),
            pl.BlockSpec((1, D), lambda i: (0, 0)),
            pl.BlockSpec((D, 1), lambda i: (0, 0)),
            pl.BlockSpec((1, 1), lambda i: (0, 0)),
        ],
        out_specs=pl.BlockSpec((1, 1, S), lambda i: (i, 0, 0)),
        out_shape=jax.ShapeDtypeStruct((G, 1, S), jnp.float32),
        compiler_params=pltpu.CompilerParams(
            dimension_semantics=("parallel",)),
    )(pos_pad, an2d, batch2d, M, W1b, W2.astype(jnp.bfloat16),
      b2.reshape(1, D), W3, b3.reshape(1, 1))

    out = pl.pallas_call(
        _epilogue_body,
        out_shape=jax.ShapeDtypeStruct((1, S), jnp.float32),
    )(partials)

    return out.reshape(S)


# speedup vs baseline: 1.0202x; 1.0202x over previous
import functools

import jax
import jax.numpy as jnp
from jax.experimental import pallas as pl
from jax.experimental.pallas import tpu as pltpu

N = 50000
D = 256
NZ_PAD = 128
S = 1024
BN = 2000
G = N // BN


def _main_body(emb_ref, w1a_ref, b1_ref, pos_ref, an_ref, batch_ref,
               w1b_ref, w2_ref, b2_ref, w3_ref, b3_ref, out_ref, m_ref):
    i = pl.program_id(0)

    @pl.when(i == 0)
    def _compute_m():
        m_ref[...] = (
            jnp.dot(emb_ref[...], w1a_ref[...],
                    preferred_element_type=jnp.float32)
            + b1_ref[...]
        ).astype(jnp.bfloat16)

    an = an_ref[...]
    onehot_an = (an == jax.lax.broadcasted_iota(jnp.int32, (1, NZ_PAD), 1)
                 ).astype(jnp.bfloat16)
    pre1 = (
        jnp.dot(onehot_an, m_ref[...], preferred_element_type=jnp.float32)
        + jnp.dot(pos_ref[...], w1b_ref[...], preferred_element_type=jnp.float32)
    )
    x1 = (0.5 * pre1) * (1.0 + jnp.tanh(0.5 * pre1))
    pre2 = jnp.dot(x1.astype(jnp.bfloat16), w2_ref[...],
                   preferred_element_type=jnp.float32) + b2_ref[...]
    x2 = (0.5 * pre2) * (1.0 + jnp.tanh(0.5 * pre2))
    e = jnp.dot(x2, w3_ref[...], preferred_element_type=jnp.float32) + b3_ref[...]

    seg = batch_ref[...]
    onehot_seg = (seg == jax.lax.broadcasted_iota(jnp.int32, (1, S), 1)
                  ).astype(jnp.float32)
    partial = jax.lax.dot_general(
        e, onehot_seg, dimension_numbers=(((0,), (0,)), ((), ())),
        preferred_element_type=jnp.float32)

    @pl.when(i == 0)
    def _init():
        out_ref[...] = partial

    @pl.when(i > 0)
    def _acc():
        out_ref[...] += partial


@functools.partial(jax.jit, static_argnames=())
def kernel(pos, emb, W1, b1, W2, b2, W3, b3, atomic_numbers, batch):
    pos_pad = jnp.pad(pos.astype(jnp.float32), ((0, 0), (0, 5)))
    emb_pad = jnp.pad(emb, ((0, NZ_PAD - emb.shape[0]), (0, 0)))
    W1a = W1[:D, :]
    W1b = jnp.pad(W1[D:, :], ((0, 5), (0, 0)))
    an2d = atomic_numbers.astype(jnp.int32).reshape(N, 1)
    batch2d = batch.astype(jnp.int32).reshape(N, 1)

    out = pl.pallas_call(
        _main_body,
        grid=(G,),
        in_specs=[
            pl.BlockSpec((NZ_PAD, D), lambda i: (0, 0)),
            pl.BlockSpec((D, D), lambda i: (0, 0)),
            pl.BlockSpec((1, D), lambda i: (0, 0)),
            pl.BlockSpec((BN, 8), lambda i: (i, 0)),
            pl.BlockSpec((BN, 1), lambda i: (i, 0)),
            pl.BlockSpec((BN, 1), lambda i: (i, 0)),
            pl.BlockSpec((8, D), lambda i: (0, 0)),
            pl.BlockSpec((D, D), lambda i: (0, 0)),
            pl.BlockSpec((1, D), lambda i: (0, 0)),
            pl.BlockSpec((D, 1), lambda i: (0, 0)),
            pl.BlockSpec((1, 1), lambda i: (0, 0)),
        ],
        out_specs=pl.BlockSpec((1, S), lambda i: (0, 0)),
        out_shape=jax.ShapeDtypeStruct((1, S), jnp.float32),
        scratch_shapes=[pltpu.VMEM((NZ_PAD, D), jnp.bfloat16)],
    )(emb_pad, W1a, b1.reshape(1, D), pos_pad, an2d, batch2d, W1b,
      W2.astype(jnp.bfloat16), b2.reshape(1, D), W3, b3.reshape(1, 1))

    return out.reshape(S)


# speedup vs baseline: 1.0830x; 1.0615x over previous
import functools

import jax
import jax.numpy as jnp
from jax.experimental import pallas as pl
from jax.experimental.pallas import tpu as pltpu

N = 50000
D = 256
NZ_PAD = 128
S = 1024
SHI = 8
SLO = 128
BN = 5000
G = N // BN


def _main_body(emb_ref, w1a_ref, b1_ref, pos_ref, an_ref, batch_ref,
               w1b_ref, w2_ref, b2_ref, w3_ref, b3_ref, out_ref, m_ref):
    i = pl.program_id(0)

    @pl.when(i == 0)
    def _compute_m():
        m_ref[...] = (
            jnp.dot(emb_ref[...], w1a_ref[...],
                    preferred_element_type=jnp.float32)
            + b1_ref[...]
        ).astype(jnp.bfloat16)

    an = an_ref[...]
    onehot_an = (an == jax.lax.broadcasted_iota(jnp.int32, (1, NZ_PAD), 1)
                 ).astype(jnp.bfloat16)
    pre1 = (
        jnp.dot(onehot_an, m_ref[...], preferred_element_type=jnp.float32)
        + jnp.dot(pos_ref[...], w1b_ref[...], preferred_element_type=jnp.float32)
    )
    x1 = (0.5 * pre1) * (1.0 + jnp.tanh(0.5 * pre1))
    pre2 = jnp.dot(x1.astype(jnp.bfloat16), w2_ref[...],
                   preferred_element_type=jnp.float32) + b2_ref[...]
    x2 = (0.5 * pre2) * (1.0 + jnp.tanh(0.5 * pre2))
    e = (jnp.dot(x2.astype(jnp.bfloat16), w3_ref[...],
                 preferred_element_type=jnp.float32)
         + b3_ref[...]).astype(jnp.bfloat16)

    seg = batch_ref[...]
    hi = jax.lax.shift_right_logical(seg, 7)
    lo = jax.lax.bitwise_and(seg, 127)
    oh_hi = (hi == jax.lax.broadcasted_iota(jnp.int32, (1, SHI), 1)
             ).astype(jnp.bfloat16)
    oh_lo = (lo == jax.lax.broadcasted_iota(jnp.int32, (1, SLO), 1)
             ).astype(jnp.bfloat16)
    weighted = oh_hi * e
    partial = jax.lax.dot_general(
        weighted, oh_lo, dimension_numbers=(((0,), (0,)), ((), ())),
        preferred_element_type=jnp.float32)

    @pl.when(i == 0)
    def _init():
        out_ref[...] = partial

    @pl.when(i > 0)
    def _acc():
        out_ref[...] += partial


@functools.partial(jax.jit, static_argnames=())
def kernel(pos, emb, W1, b1, W2, b2, W3, b3, atomic_numbers, batch):
    pos_pad = jnp.pad(pos.astype(jnp.bfloat16), ((0, 0), (0, 5)))
    emb_pad = jnp.pad(emb, ((0, NZ_PAD - emb.shape[0]), (0, 0)))
    W1a = W1[:D, :]
    W1b = jnp.pad(W1[D:, :].astype(jnp.bfloat16), ((0, 5), (0, 0)))
    an2d = atomic_numbers.astype(jnp.int32).reshape(N, 1)
    batch2d = batch.astype(jnp.int32).reshape(N, 1)

    out = pl.pallas_call(
        _main_body,
        grid=(G,),
        in_specs=[
            pl.BlockSpec((NZ_PAD, D), lambda i: (0, 0)),
            pl.BlockSpec((D, D), lambda i: (0, 0)),
            pl.BlockSpec((1, D), lambda i: (0, 0)),
            pl.BlockSpec((BN, 8), lambda i: (i, 0)),
            pl.BlockSpec((BN, 1), lambda i: (i, 0)),
            pl.BlockSpec((BN, 1), lambda i: (i, 0)),
            pl.BlockSpec((8, D), lambda i: (0, 0)),
            pl.BlockSpec((D, D), lambda i: (0, 0)),
            pl.BlockSpec((1, D), lambda i: (0, 0)),
            pl.BlockSpec((D, 1), lambda i: (0, 0)),
            pl.BlockSpec((1, 1), lambda i: (0, 0)),
        ],
        out_specs=pl.BlockSpec((SHI, SLO), lambda i: (0, 0)),
        out_shape=jax.ShapeDtypeStruct((SHI, SLO), jnp.float32),
        scratch_shapes=[pltpu.VMEM((NZ_PAD, D), jnp.bfloat16)],
    )(emb_pad, W1a, b1.reshape(1, D), pos_pad, an2d, batch2d, W1b,
      W2.astype(jnp.bfloat16), b2.reshape(1, D),
      W3.astype(jnp.bfloat16), b3.reshape(1, 1))

    return out.reshape(S)
